# baseline (speedup 1.0000x reference)
import functools

import jax
import jax.numpy as jnp
from jax import lax
from jax.experimental import pallas as pl
from jax.experimental.pallas import tpu as pltpu
from jax.experimental.pallas import tpu_sc as plsc


def _cast_body(x_ref, o_ref):
    o_ref[...] = x_ref[...].astype(jnp.int32)


@functools.lru_cache(maxsize=None)
def _build_cast(R, D, blk):
    return pl.pallas_call(
        _cast_body,
        grid=(pl.cdiv(R, blk),),
        in_specs=[pl.BlockSpec((blk, D), lambda i: (i, 0))],
        out_specs=pl.BlockSpec((blk, D), lambda i: (i, 0)),
        out_shape=jax.ShapeDtypeStruct((R, D), jnp.int32),
    )


@functools.lru_cache(maxsize=None)
def _build_lookup(V, D, B, N1):
    info = plsc.get_sparse_core_info()
    NC, NS, L = info.num_cores, info.num_subcores, info.num_lanes
    NW = NC * NS
    assert B % NW == 0 and D % L == 0
    per_w = B // NW
    GR = 2 * L
    NB = 4
    n_g = per_w // GR
    assert n_g >= NB
    mesh = plsc.VectorSubcoreMesh(core_axis_name="c", subcore_axis_name="s")

    @functools.partial(
        pl.kernel,
        mesh=mesh,
        out_type=jax.ShapeDtypeStruct((B, D), jnp.int32),
        scratch_types=[
            pltpu.VMEM((per_w,), jnp.int32),
        ]
        + [pltpu.VMEM((GR, D), jnp.int32) for _ in range(NB)]
        + [pltpu.SemaphoreType.DMA for _ in range(2 * NB)],
    )
    def lookup(x_hbm, ow_hbm, nw_hbm, out_hbm, idx_v, *scr):
        bufs = scr[:NB]
        gsems = scr[NB:2 * NB]
        ssems = scr[2 * NB:]
        wid = lax.axis_index("s") * NC + lax.axis_index("c")
        base = wid * per_w
        pltpu.sync_copy(x_hbm.at[pl.ds(base, per_w)], idx_v)

        def issue(g, buf, gsem):
            for h in range(GR // L):
                ivec = idx_v[pl.ds(g * GR + h * L, L)]
                for r in range(L):
                    iv = ivec[r]
                    good = iv < V

                    @pl.when(good)
                    def _():
                        pltpu.async_copy(
                            ow_hbm.at[iv], buf.at[h * L + r], gsem)

                    @pl.when(jnp.logical_not(good))
                    def _():
                        pltpu.async_copy(
                            nw_hbm.at[iv - (V - 1)], buf.at[h * L + r], gsem)

        for k in range(NB):
            issue(k, bufs[k], gsems[k])

        for g in range(n_g):
            b = g % NB
            pltpu.make_async_copy(
                ow_hbm.at[pl.ds(0, GR)], bufs[b], gsems[b]).wait()
            pltpu.async_copy(
                bufs[b], out_hbm.at[pl.ds(base + g * GR, GR)], ssems[b])
            if g + NB < n_g:
                pltpu.make_async_copy(
                    bufs[b], out_hbm.at[pl.ds(0, GR)], ssems[b]).wait()
                issue(g + NB, bufs[b], gsems[b])

        for g in range(max(0, n_g - NB), n_g):
            b = g % NB
            pltpu.make_async_copy(
                bufs[b], out_hbm.at[pl.ds(0, GR)], ssems[b]).wait()

    return lookup


def kernel(x, original_weight, new_weight):
    V, D = original_weight.shape
    N1 = new_weight.shape[0]
    Bt, S = x.shape
    B = Bt * S
    ow_i32 = _build_cast(V, D, 512)(original_weight)
    nw_i32 = _build_cast(N1, D, N1)(new_weight)
    lookup = _build_lookup(V, D, B, N1)
    out = lookup(x.reshape(B), ow_i32, nw_i32)
    return out.reshape(Bt, S, D)


# speedup vs baseline: 1.8537x; 1.8537x over previous
import functools

import jax
import jax.numpy as jnp
from jax import lax
from jax.experimental import pallas as pl
from jax.experimental.pallas import tpu as pltpu
from jax.experimental.pallas import tpu_sc as plsc


@functools.lru_cache(maxsize=None)
def _build_lookup(V, D, B, N1):
    info = plsc.get_sparse_core_info()
    NC, NS, L = info.num_cores, info.num_subcores, info.num_lanes
    NW = NC * NS
    assert B % NW == 0 and D % L == 0
    per_w = B // NW
    GR = L
    n_g = per_w // GR
    assert n_g % 2 == 0
    mesh = plsc.VectorSubcoreMesh(core_axis_name="c", subcore_axis_name="s")

    @functools.partial(
        pl.kernel,
        mesh=mesh,
        out_type=jax.ShapeDtypeStruct((B, D), jnp.int32),
        scratch_types=[
            pltpu.VMEM((per_w,), jnp.int32),
            pltpu.VMEM((GR, D), jnp.float32),
            pltpu.VMEM((GR, D), jnp.float32),
            pltpu.VMEM((GR, D), jnp.int32),
            pltpu.VMEM((GR, D), jnp.int32),
            pltpu.SemaphoreType.DMA,
            pltpu.SemaphoreType.DMA,
            pltpu.SemaphoreType.DMA,
            pltpu.SemaphoreType.DMA,
        ],
    )
    def lookup(x_hbm, ow_hbm, nw_hbm, out_hbm,
               idx_v, buf0, buf1, outb0, outb1,
               gsem0, gsem1, osem0, osem1):
        wid = lax.axis_index("s") * NC + lax.axis_index("c")
        base = wid * per_w
        pltpu.sync_copy(x_hbm.at[pl.ds(base, per_w)], idx_v)

        def issue_row(ivec, r, buf, gsem):
            iv = ivec[r]
            good = iv < V

            @pl.when(good)
            def _():
                pltpu.async_copy(ow_hbm.at[iv], buf.at[r], gsem)

            @pl.when(jnp.logical_not(good))
            def _():
                pltpu.async_copy(nw_hbm.at[iv - (V - 1)], buf.at[r], gsem)

        def issue(g, buf, gsem):
            ivec = idx_v[pl.ds(g * GR, GR)]
            for r in range(GR):
                issue_row(ivec, r, buf, gsem)

        def wait_rows(buf, gsem):
            pltpu.make_async_copy(ow_hbm.at[pl.ds(0, GR)], buf, gsem).wait()

        def convert_and_issue(g, buf, outb, gsem, do_issue):
            if do_issue:
                ivec = idx_v[pl.ds((g + 2) * GR, GR)]
            for r in range(GR):
                for c in range(D // L):
                    cs = pl.ds(c * L, L)
                    outb[r, cs] = buf[r, cs].astype(jnp.int32)
                if do_issue:
                    issue_row(ivec, r, buf, gsem)

        def wait_store(outb, osem):
            pltpu.make_async_copy(outb, out_hbm.at[pl.ds(0, GR)], osem).wait()

        def half(i, g, buf, outb, gsem, osem, do_issue):
            wait_rows(buf, gsem)

            @pl.when(i >= 1)
            def _():
                wait_store(outb, osem)

            convert_and_issue(g, buf, outb, gsem, do_issue)
            pltpu.async_copy(outb, out_hbm.at[pl.ds(base + g * GR, GR)], osem)

        issue(0, buf0, gsem0)
        issue(1, buf1, gsem1)

        def pair_body(i, _):
            half(i, 2 * i, buf0, outb0, gsem0, osem0, True)
            half(i, 2 * i + 1, buf1, outb1, gsem1, osem1, True)
            return 0

        lax.fori_loop(0, n_g // 2 - 1, pair_body, 0)

        g_last = n_g - 2
        wait_rows(buf0, gsem0)
        wait_store(outb0, osem0)
        convert_and_issue(g_last, buf0, outb0, gsem0, False)
        pltpu.async_copy(outb0, out_hbm.at[pl.ds(base + g_last * GR, GR)],
                         osem0)
        wait_rows(buf1, gsem1)
        wait_store(outb1, osem1)
        convert_and_issue(g_last + 1, buf1, outb1, gsem1, False)
        pltpu.async_copy(
            outb1, out_hbm.at[pl.ds(base + (g_last + 1) * GR, GR)], osem1)
        pltpu.make_async_copy(outb0, out_hbm.at[pl.ds(0, GR)], osem0).wait()
        pltpu.make_async_copy(outb1, out_hbm.at[pl.ds(0, GR)], osem1).wait()

    return lookup


def kernel(x, original_weight, new_weight):
    V, D = original_weight.shape
    N1 = new_weight.shape[0]
    Bt, S = x.shape
    B = Bt * S
    lookup = _build_lookup(V, D, B, N1)
    out = lookup(x.reshape(B), original_weight, new_weight)
    return out.reshape(Bt, S, D)
